# baseline (speedup 1.0000x reference)
import dataclasses
import functools

import jax
import jax.numpy as jnp
from jax import lax
from jax.experimental import pallas as pl
from jax.experimental.pallas import tpu as pltpu
from jax.experimental.pallas import tpu_sc as plsc

_ROUTER_DIM = 256
_NUM_EXPERTS = 64
_TOP_K = 8
_BLK = 1024
_CHUNK_SIZES = (24576, 8192)

_SC_CORES = 2
_SC_SUBCORES = 16
_SC_WORKERS = _SC_CORES * _SC_SUBCORES
_LANES = 16


def _logits_kernel(x1_ref, x2_ref, wt1_ref, wt2_ref, b_in_ref, b_mat_ref,
                   wg_ref, b_gate_ref, out_ref):
    xp = jnp.dot(x1_ref[...], wt1_ref[...], preferred_element_type=jnp.float32)
    xp = xp + jnp.dot(x2_ref[...], wt2_ref[...],
                      preferred_element_type=jnp.float32)
    xp = xp + b_in_ref[...]
    h = jnp.tanh(jnp.dot(xp, b_mat_ref[...],
                         preferred_element_type=jnp.float32)) * 0.1
    logits = jnp.dot(h, wg_ref[...], preferred_element_type=jnp.float32)
    out_ref[...] = logits + b_gate_ref[...]


def _tc_logits_chunk(x, wt, b_in2, B, wg, b_gate2, chunk_rows, base):
    hidden = x.shape[1]
    blocks = chunk_rows // _BLK
    return pl.pallas_call(
        _logits_kernel,
        grid=(blocks,),
        in_specs=[
            pl.BlockSpec((_BLK, hidden // 2), lambda i: (base + i, 0)),
            pl.BlockSpec((_BLK, hidden // 2), lambda i: (base + i, 1)),
            pl.BlockSpec((hidden // 2, _ROUTER_DIM), lambda i: (0, 0)),
            pl.BlockSpec((hidden // 2, _ROUTER_DIM), lambda i: (1, 0)),
            pl.BlockSpec((1, _ROUTER_DIM), lambda i: (0, 0)),
            pl.BlockSpec((_ROUTER_DIM, _ROUTER_DIM), lambda i: (0, 0)),
            pl.BlockSpec((_ROUTER_DIM, _NUM_EXPERTS), lambda i: (0, 0)),
            pl.BlockSpec((1, _NUM_EXPERTS), lambda i: (0, 0)),
        ],
        out_specs=pl.BlockSpec((_BLK, _NUM_EXPERTS), lambda i: (i, 0)),
        out_shape=jax.ShapeDtypeStruct((chunk_rows, _NUM_EXPERTS),
                                       jnp.float32),
        compiler_params=pltpu.CompilerParams(
            dimension_semantics=("parallel",),
        ),
    )(x, x, wt, wt, b_in2, B, wg, b_gate2)


def _sc_topk(logits):
    n = logits.shape[0]
    rows_per_worker = n // _SC_WORKERS
    rows_per_chunk = next(d for d in range(min(256, rows_per_worker), 0, -1)
                          if rows_per_worker % d == 0)
    n_chunks = rows_per_worker // rows_per_chunk
    mesh = plsc.VectorSubcoreMesh(core_axis_name="c", subcore_axis_name="s",
                                  num_cores=_SC_CORES,
                                  num_subcores=_SC_SUBCORES)
    cp = pltpu.CompilerParams()
    if "needs_layout_passes" in pltpu.CompilerParams.__dataclass_fields__:
        cp = dataclasses.replace(cp, needs_layout_passes=False)

    @functools.partial(
        pl.kernel, mesh=mesh,
        out_type=(jax.ShapeDtypeStruct((n, _LANES), jnp.int32),
                  jax.ShapeDtypeStruct((n, _LANES), jnp.float32)),
        scratch_types=[
            pltpu.VMEM((rows_per_chunk, _NUM_EXPERTS), jnp.float32),
            pltpu.VMEM((rows_per_chunk, _LANES), jnp.int32),
            pltpu.VMEM((rows_per_chunk, _LANES), jnp.float32),
        ],
        compiler_params=cp,
    )
    def k(l_hbm, idx_hbm, w_hbm, l_v, i_v, w_v):
        wid = lax.axis_index("s") * _SC_CORES + lax.axis_index("c")
        iota16 = lax.iota(jnp.int32, _LANES)
        lane_lt8 = iota16 < 8

        @pl.loop(0, n_chunks)
        def _(ci):
            base = wid * rows_per_worker + ci * rows_per_chunk
            pltpu.sync_copy(l_hbm.at[pl.ds(base, rows_per_chunk)], l_v)

            @pl.loop(0, rows_per_chunk)
            def _(r):
                merged = []
                for c in range(4):
                    v = l_v[r, pl.ds(c * _LANES, _LANES)]
                    y = lax.bitcast_convert_type(v, jnp.int32)
                    s = y ^ ((y >> 31) & jnp.int32(0x7FFFFFFF))
                    key = (s & jnp.int32(~63)) | (63 - (iota16 + c * _LANES))
                    sk, _unused = plsc.sort_key_val(key, key, descending=True)
                    merged.append(sk)

                def merge(a, b):
                    comb = jnp.where(lane_lt8, a, lax.rev(b, (0,)))
                    sk, _unused = plsc.sort_key_val(comb, comb,
                                                    descending=True)
                    return sk

                fin = merge(merge(merged[0], merged[1]),
                            merge(merged[2], merged[3]))
                idx = 63 - (fin & 63)
                y2 = fin & jnp.int32(~63)
                y2 = y2 ^ ((y2 >> 31) & jnp.int32(0x7FFFFFFF))
                tv = lax.bitcast_convert_type(y2, jnp.float32)
                m = jnp.max(tv)
                e = jnp.where(lane_lt8, jnp.exp(tv - m), 0.0)
                w = e / jnp.sum(e)
                i_v[r, :] = idx
                w_v[r, :] = w

            pltpu.sync_copy(i_v, idx_hbm.at[pl.ds(base, rows_per_chunk)])
            pltpu.sync_copy(w_v, w_hbm.at[pl.ds(base, rows_per_chunk)])

    return k(logits)


@jax.jit
def kernel(x, W_in, b_in, tau, A, B, W_gate, b_gate):
    del tau, A
    n_tokens = x.shape[0]
    wt = W_in.T
    wg = W_gate.T
    b_in2 = b_in.reshape(1, _ROUTER_DIM)
    b_gate2 = b_gate.reshape(1, _NUM_EXPERTS)
    del n_tokens
    idx_parts = []
    w_parts = []
    row_base = 0
    for chunk_rows in _CHUNK_SIZES:
        logits_c = _tc_logits_chunk(x, wt, b_in2, B, wg, b_gate2,
                                    chunk_rows, row_base // _BLK)
        idx16, w16 = _sc_topk(logits_c)
        idx_parts.append(idx16[:, :_TOP_K])
        w_parts.append(w16[:, :_TOP_K])
        row_base += chunk_rows
    return (jnp.concatenate(idx_parts, axis=0),
            jnp.concatenate(w_parts, axis=0))


# speedup vs baseline: 1.0244x; 1.0244x over previous
import dataclasses
import functools

import jax
import jax.numpy as jnp
from jax import lax
from jax.experimental import pallas as pl
from jax.experimental.pallas import tpu as pltpu
from jax.experimental.pallas import tpu_sc as plsc

_ROUTER_DIM = 256
_NUM_EXPERTS = 64
_TOP_K = 8
_BLK = 1024
_CHUNK_SIZES = (16384, 16384)

_SC_CORES = 2
_SC_SUBCORES = 16
_SC_WORKERS = _SC_CORES * _SC_SUBCORES
_LANES = 16


def _logits_kernel(x1_ref, x2_ref, wt1_ref, wt2_ref, b_in_ref, b_mat_ref,
                   wg_ref, b_gate_ref, out_ref):
    xp = jnp.dot(x1_ref[...], wt1_ref[...], preferred_element_type=jnp.float32)
    xp = xp + jnp.dot(x2_ref[...], wt2_ref[...],
                      preferred_element_type=jnp.float32)
    xp = xp + b_in_ref[...]
    h = jnp.tanh(jnp.dot(xp, b_mat_ref[...],
                         preferred_element_type=jnp.float32)) * 0.1
    logits = jnp.dot(h, wg_ref[...], preferred_element_type=jnp.float32)
    out_ref[...] = logits + b_gate_ref[...]


def _tc_logits_chunk(x, wt, b_in2, B, wg, b_gate2, chunk_rows, base):
    hidden = x.shape[1]
    blocks = chunk_rows // _BLK
    return pl.pallas_call(
        _logits_kernel,
        grid=(blocks,),
        in_specs=[
            pl.BlockSpec((_BLK, hidden // 2), lambda i: (base + i, 0)),
            pl.BlockSpec((_BLK, hidden // 2), lambda i: (base + i, 1)),
            pl.BlockSpec((hidden // 2, _ROUTER_DIM), lambda i: (0, 0)),
            pl.BlockSpec((hidden // 2, _ROUTER_DIM), lambda i: (1, 0)),
            pl.BlockSpec((1, _ROUTER_DIM), lambda i: (0, 0)),
            pl.BlockSpec((_ROUTER_DIM, _ROUTER_DIM), lambda i: (0, 0)),
            pl.BlockSpec((_ROUTER_DIM, _NUM_EXPERTS), lambda i: (0, 0)),
            pl.BlockSpec((1, _NUM_EXPERTS), lambda i: (0, 0)),
        ],
        out_specs=pl.BlockSpec((_BLK, _NUM_EXPERTS), lambda i: (i, 0)),
        out_shape=jax.ShapeDtypeStruct((chunk_rows, _NUM_EXPERTS),
                                       jnp.float32),
        compiler_params=pltpu.CompilerParams(
            dimension_semantics=("parallel",),
        ),
    )(x, x, wt, wt, b_in2, B, wg, b_gate2)


def _sc_topk(logits):
    n = logits.shape[0]
    rows_per_worker = n // _SC_WORKERS
    rows_per_chunk = next(d for d in range(min(256, rows_per_worker), 0, -1)
                          if rows_per_worker % d == 0)
    n_chunks = rows_per_worker // rows_per_chunk
    mesh = plsc.VectorSubcoreMesh(core_axis_name="c", subcore_axis_name="s",
                                  num_cores=_SC_CORES,
                                  num_subcores=_SC_SUBCORES)
    cp = pltpu.CompilerParams()
    if "needs_layout_passes" in pltpu.CompilerParams.__dataclass_fields__:
        cp = dataclasses.replace(cp, needs_layout_passes=False)

    @functools.partial(
        pl.kernel, mesh=mesh,
        out_type=(jax.ShapeDtypeStruct((n, _LANES), jnp.int32),
                  jax.ShapeDtypeStruct((n, _LANES), jnp.float32)),
        scratch_types=[
            pltpu.VMEM((rows_per_chunk, _NUM_EXPERTS), jnp.float32),
            pltpu.VMEM((rows_per_chunk, _LANES), jnp.int32),
            pltpu.VMEM((rows_per_chunk, _LANES), jnp.float32),
        ],
        compiler_params=cp,
    )
    def k(l_hbm, idx_hbm, w_hbm, l_v, i_v, w_v):
        wid = lax.axis_index("s") * _SC_CORES + lax.axis_index("c")
        iota16 = lax.iota(jnp.int32, _LANES)
        lane_lt8 = iota16 < 8

        @pl.loop(0, n_chunks)
        def _(ci):
            base = wid * rows_per_worker + ci * rows_per_chunk
            pltpu.sync_copy(l_hbm.at[pl.ds(base, rows_per_chunk)], l_v)

            @pl.loop(0, rows_per_chunk)
            def _(r):
                merged = []
                for c in range(4):
                    v = l_v[r, pl.ds(c * _LANES, _LANES)]
                    y = lax.bitcast_convert_type(v, jnp.int32)
                    s = y ^ ((y >> 31) & jnp.int32(0x7FFFFFFF))
                    key = (s & jnp.int32(~63)) | (63 - (iota16 + c * _LANES))
                    sk, _unused = plsc.sort_key_val(key, key, descending=True)
                    merged.append(sk)

                def merge(a, b):
                    comb = jnp.where(lane_lt8, a, lax.rev(b, (0,)))
                    sk, _unused = plsc.sort_key_val(comb, comb,
                                                    descending=True)
                    return sk

                fin = merge(merge(merged[0], merged[1]),
                            merge(merged[2], merged[3]))
                idx = 63 - (fin & 63)
                y2 = fin & jnp.int32(~63)
                y2 = y2 ^ ((y2 >> 31) & jnp.int32(0x7FFFFFFF))
                tv = lax.bitcast_convert_type(y2, jnp.float32)
                m = jnp.max(tv)
                e = jnp.where(lane_lt8, jnp.exp(tv - m), 0.0)
                w = e / jnp.sum(e)
                i_v[r, :] = idx
                w_v[r, :] = w

            pltpu.sync_copy(i_v, idx_hbm.at[pl.ds(base, rows_per_chunk)])
            pltpu.sync_copy(w_v, w_hbm.at[pl.ds(base, rows_per_chunk)])

    return k(logits)


@jax.jit
def kernel(x, W_in, b_in, tau, A, B, W_gate, b_gate):
    del tau, A
    n_tokens = x.shape[0]
    wt = W_in.T
    wg = W_gate.T
    b_in2 = b_in.reshape(1, _ROUTER_DIM)
    b_gate2 = b_gate.reshape(1, _NUM_EXPERTS)
    del n_tokens
    idx_parts = []
    w_parts = []
    row_base = 0
    for chunk_rows in _CHUNK_SIZES:
        logits_c = _tc_logits_chunk(x, wt, b_in2, B, wg, b_gate2,
                                    chunk_rows, row_base // _BLK)
        idx16, w16 = _sc_topk(logits_c)
        idx_parts.append(idx16[:, :_TOP_K])
        w_parts.append(w16[:, :_TOP_K])
        row_base += chunk_rows
    return (jnp.concatenate(idx_parts, axis=0),
            jnp.concatenate(w_parts, axis=0))
